# baseline (speedup 1.0000x reference)
import functools

import jax
import jax.numpy as jnp
from jax import lax
from jax.experimental import pallas as pl
from jax.experimental.pallas import tpu as pltpu
from jax.experimental.pallas import tpu_sc as plsc

N = 10000
E = 320000
NHID = 128
NCORE = 2
NSUB = 16
CHUNK = 128
NW = NCORE * NSUB
C0 = 80
C1 = 80
TOTAL_CHUNKS = NSUB * (C0 + C1)
E_PAD = TOTAL_CHUNKS * CHUNK
ROWS_PER_TILE = 640
ACC_ROWS = ROWS_PER_TILE * NSUB

_sc_mesh = plsc.VectorSubcoreMesh(core_axis_name="c", subcore_axis_name="s")


@functools.partial(
    pl.kernel,
    out_type=(
        jax.ShapeDtypeStruct((ACC_ROWS, NHID), jnp.float32),
        jax.ShapeDtypeStruct((ACC_ROWS, NHID), jnp.float32),
    ),
    mesh=_sc_mesh,
    scratch_types=[
        pltpu.VMEM((C0, CHUNK), jnp.int32),
        pltpu.VMEM((2, CHUNK), jnp.int32),
        pltpu.VMEM((4, CHUNK), jnp.int32),
        pltpu.VMEM((CHUNK, NHID), jnp.float32),
        pltpu.VMEM((CHUNK, NHID), jnp.float32),
        pltpu.VMEM_SHARED((ACC_ROWS, NHID), jnp.float32),
        pltpu.SemaphoreType.DMA,
        pltpu.SemaphoreType.DMA,
        pltpu.SemaphoreType.DMA,
        pltpu.SemaphoreType.DMA,
    ],
)
def _sc_aggregate(h_hbm, pk_hbm, out0_hbm, out1_hbm,
                  pk_v, sstage, dstage, bf0, bf1, acc_sh,
                  g0, g1, s0, s1):
    cid = lax.axis_index("c")
    sid = lax.axis_index("s")
    wid = cid * NSUB + sid
    base = sid * ROWS_PER_TILE
    bufs = (bf0, bf1)
    gsem = (g0, g1)
    ssem = (s0, s1)

    zeros = jnp.zeros((16,), jnp.float32)

    def _zero_row(r, carry):
        for c in range(NHID // 16):
            bf0[r, pl.ds(c * 16, 16)] = zeros
        return carry

    with jax.named_scope("idx_fire"):
        @pl.when(cid == 0)
        def _():
            pltpu.async_copy(pk_hbm.at[pl.ds(sid * C0, C0)], pk_v, g0)

        @pl.when(cid == 1)
        def _():
            pltpu.async_copy(pk_hbm.at[pl.ds(NSUB * C0 + sid * C1, C1)],
                             pk_v.at[pl.ds(0, C1)], g0)

    with jax.named_scope("zero_acc"):
        lax.fori_loop(0, CHUNK, _zero_row, 0)

        @pl.when(cid == 0)
        def _():
            @pl.when(sid < NSUB - 1)
            def _():
                pltpu.sync_copy(h_hbm.at[pl.ds(base, ROWS_PER_TILE)],
                                acc_sh.at[pl.ds(base, ROWS_PER_TILE)])

            @pl.when(sid == NSUB - 1)
            def _():
                _live = N - (NSUB - 1) * ROWS_PER_TILE
                pltpu.sync_copy(h_hbm.at[pl.ds(base, _live)],
                                acc_sh.at[pl.ds(base, _live)])
                pltpu.sync_copy(bf0, acc_sh.at[pl.ds(base + _live, CHUNK)])
                pltpu.sync_copy(bf0.at[pl.ds(0, ACC_ROWS - N - CHUNK)],
                                acc_sh.at[pl.ds(N + CHUNK, ACC_ROWS - N - CHUNK)])

        @pl.when(cid == 1)
        def _():
            for k in range(ROWS_PER_TILE // CHUNK):
                pltpu.sync_copy(bf0, acc_sh.at[pl.ds(base + k * CHUNK, CHUNK)])
            _tail_rows = ROWS_PER_TILE % CHUNK
            if _tail_rows:
                _full = (ROWS_PER_TILE // CHUNK) * CHUNK
                pltpu.sync_copy(bf0.at[pl.ds(0, _tail_rows)],
                                acc_sh.at[pl.ds(base + _full, _tail_rows)])

    with jax.named_scope("idx_wait"):
        @pl.when(cid == 0)
        def _():
            pltpu.make_async_copy(pk_hbm.at[pl.ds(sid * C0, C0)],
                                  pk_v, g0).wait()

        @pl.when(cid == 1)
        def _():
            pltpu.make_async_copy(pk_hbm.at[pl.ds(NSUB * C0 + sid * C1, C1)],
                                  pk_v.at[pl.ds(0, C1)], g0).wait()

        plsc.subcore_barrier()

    def _unpack(j, r2, r4):
        for q in range(CHUNK // 16):
            w = pk_v[j, pl.ds(q * 16, 16)]
            sstage[r2, pl.ds(q * 16, 16)] = w & 0xFFFF
            dstage[r4, pl.ds(q * 16, 16)] = lax.shift_right_logical(w, 16)

    def _fire_g(r, b):
        pltpu.async_copy(h_hbm.at[sstage.at[r]], bufs[b], gsem[b])

    def _wait_g(r, b):
        pltpu.make_async_copy(h_hbm.at[sstage.at[r]], bufs[b], gsem[b]).wait()

    def _fire_s(r, b):
        pltpu.async_copy(bufs[b], acc_sh.at[dstage.at[r]], ssem[b], add=True)

    def _wait_s(r, b):
        pltpu.make_async_copy(bufs[b], acc_sh.at[dstage.at[r]], ssem[b]).wait()

    def _emit(j, m2, m4, first, fire_next_g):
        n2 = (m2 + 1) % 2
        if fire_next_g:
            _unpack(j + 1, n2, (m4 + 1) % 4)
            if not first:
                _wait_s((m4 + 3) % 4, n2)
            _fire_g(n2, n2)
        _wait_g(m2, m2)
        _fire_s(m4, m2)

    def _run(nc):
        _unpack(0, 0, 0)
        _fire_g(0, 0)
        _emit(0, 0, 0, True, True)

        steady_n = nc - 2
        loop_n = steady_n // 4
        tail = steady_n % 4

        if loop_n > 0:
            def _steady(g, carry):
                j = 1 + 4 * g
                for k in range(4):
                    _emit(j + k, (1 + k) % 2, (1 + k) % 4, False, True)
                return carry

            lax.fori_loop(0, loop_n, _steady, 0)
        for t in range(tail):
            j = 1 + 4 * loop_n + t
            _emit(j, j % 2, j % 4, False, True)

        j = nc - 1
        _emit(j, j % 2, j % 4, False, False)
        _wait_s((nc - 2) % 4, (nc - 2) % 2)
        _wait_s((nc - 1) % 4, (nc - 1) % 2)

    with jax.named_scope("edge_pipe"):
        @pl.when(cid == 0)
        def _():
            _run(C0)

        @pl.when(cid == 1)
        def _():
            _run(C1)

        plsc.subcore_barrier()

    with jax.named_scope("out_copy"):
        @pl.when(cid == 0)
        def _():
            pltpu.sync_copy(acc_sh.at[pl.ds(base, ROWS_PER_TILE)],
                            out0_hbm.at[pl.ds(base, ROWS_PER_TILE)])

        @pl.when(cid == 1)
        def _():
            pltpu.sync_copy(acc_sh.at[pl.ds(base, ROWS_PER_TILE)],
                            out1_hbm.at[pl.ds(base, ROWS_PER_TILE)])


ER = E // CHUNK
_PBLK = 1280


def _pack_body(s_ref, d_ref, o_ref):
    i = pl.program_id(0)
    grow = i * _PBLK + lax.broadcasted_iota(jnp.int32, (_PBLK, CHUNK), 0)
    col = lax.broadcasted_iota(jnp.int32, (_PBLK, CHUNK), 1)
    pe = (grow - ER) * CHUNK + col
    pad_val = pe | ((N + pe % (ACC_ROWS - N)) << 16)
    real_val = s_ref[0] | (d_ref[0] << 16)
    o_ref[...] = jnp.where(grow < ER, real_val, pad_val)


_pack = pl.pallas_call(
    _pack_body,
    grid=(TOTAL_CHUNKS // _PBLK,),
    in_specs=[
        pl.BlockSpec((1, _PBLK, CHUNK), lambda i: (0, i, 0)),
        pl.BlockSpec((1, _PBLK, CHUNK), lambda i: (1, i, 0)),
    ],
    out_specs=pl.BlockSpec((_PBLK, CHUNK), lambda i: (i, 0)),
    out_shape=jax.ShapeDtypeStruct((TOTAL_CHUNKS, CHUNK), jnp.int32),
)


def _mlp_body(a0_ref, a1_ref, w1_ref, b1_ref, w2_ref, b2_ref, o_ref):
    h = a0_ref[...] + a1_ref[...]
    y = jnp.dot(h, w1_ref[...], preferred_element_type=jnp.float32) + b1_ref[...]
    y = jnp.maximum(y, 0.0)
    z = jnp.dot(y, w2_ref[...], preferred_element_type=jnp.float32) + b2_ref[...]
    o_ref[...] = jnp.maximum(z, 0.0)


_BLK = 2000
_mlp = pl.pallas_call(
    _mlp_body,
    grid=(N // _BLK,),
    in_specs=[
        pl.BlockSpec((_BLK, NHID), lambda i: (i, 0)),
        pl.BlockSpec((_BLK, NHID), lambda i: (i, 0)),
        pl.BlockSpec((NHID, NHID), lambda i: (0, 0)),
        pl.BlockSpec((1, NHID), lambda i: (0, 0)),
        pl.BlockSpec((NHID, NHID), lambda i: (0, 0)),
        pl.BlockSpec((1, NHID), lambda i: (0, 0)),
    ],
    out_specs=pl.BlockSpec((_BLK, NHID), lambda i: (i, 0)),
    out_shape=jax.ShapeDtypeStruct((N, NHID), jnp.float32),
)


def kernel(x, edge_index, W1_0, b1_0, W2_0, b2_0, W1_1, b1_1, W2_1, b2_1):
    ei3 = edge_index.reshape(2, ER, CHUNK)
    packed = _pack(ei3, ei3)

    h = x
    for (W1, b1, W2, b2) in ((W1_0, b1_0, W2_0, b2_0), (W1_1, b1_1, W2_1, b2_1)):
        a0, a1 = _sc_aggregate(h, packed)
        h = _mlp(a0, a1, W1, b1.reshape(1, NHID), W2, b2.reshape(1, NHID))
    return h


# speedup vs baseline: 1.0189x; 1.0189x over previous
import functools

import jax
import jax.numpy as jnp
from jax import lax
from jax.experimental import pallas as pl
from jax.experimental.pallas import tpu as pltpu
from jax.experimental.pallas import tpu_sc as plsc

N = 10000
E = 320000
NHID = 128
NCORE = 2
NSUB = 16
CHUNK = 128
NW = NCORE * NSUB
C0 = 80
C1 = 80
TOTAL_CHUNKS = NSUB * (C0 + C1)
E_PAD = TOTAL_CHUNKS * CHUNK
ROWS_PER_TILE = 640
ACC_ROWS = ROWS_PER_TILE * NSUB

_sc_mesh = plsc.VectorSubcoreMesh(core_axis_name="c", subcore_axis_name="s")


@functools.partial(
    pl.kernel,
    out_type=(
        jax.ShapeDtypeStruct((ACC_ROWS, NHID), jnp.float32),
        jax.ShapeDtypeStruct((ACC_ROWS, NHID), jnp.float32),
    ),
    mesh=_sc_mesh,
    scratch_types=[
        pltpu.VMEM((C0, CHUNK), jnp.int32),
        pltpu.VMEM((2, CHUNK), jnp.int32),
        pltpu.VMEM((4, CHUNK), jnp.int32),
        pltpu.VMEM((CHUNK, NHID), jnp.float32),
        pltpu.VMEM((CHUNK, NHID), jnp.float32),
        pltpu.VMEM_SHARED((ACC_ROWS, NHID), jnp.float32),
        pltpu.SemaphoreType.DMA,
        pltpu.SemaphoreType.DMA,
        pltpu.SemaphoreType.DMA,
        pltpu.SemaphoreType.DMA,
    ],
)
def _sc_aggregate(h_hbm, pk_hbm, out0_hbm, out1_hbm,
                  pk_v, sstage, dstage, bf0, bf1, acc_sh,
                  g0, g1, s0, s1):
    cid = lax.axis_index("c")
    sid = lax.axis_index("s")
    wid = cid * NSUB + sid
    base = sid * ROWS_PER_TILE
    bufs = (bf0, bf1)
    gsem = (g0, g1)
    ssem = (s0, s1)

    zeros = jnp.zeros((16,), jnp.float32)

    def _zero_row(r, carry):
        for c in range(NHID // 16):
            bf0[r, pl.ds(c * 16, 16)] = zeros
        return carry

    with jax.named_scope("idx_fire"):
        @pl.when(cid == 0)
        def _():
            pltpu.async_copy(pk_hbm.at[pl.ds(sid * C0, C0)], pk_v, g0)

        @pl.when(cid == 1)
        def _():
            pltpu.async_copy(pk_hbm.at[pl.ds(NSUB * C0 + sid * C1, C1)],
                             pk_v.at[pl.ds(0, C1)], g0)

    with jax.named_scope("zero_acc"):
        lax.fori_loop(0, CHUNK, _zero_row, 0)
        for k in range(ROWS_PER_TILE // CHUNK):
            pltpu.sync_copy(bf0, acc_sh.at[pl.ds(base + k * CHUNK, CHUNK)])
        _tail_rows = ROWS_PER_TILE % CHUNK
        if _tail_rows:
            _full = (ROWS_PER_TILE // CHUNK) * CHUNK
            pltpu.sync_copy(bf0.at[pl.ds(0, _tail_rows)],
                            acc_sh.at[pl.ds(base + _full, _tail_rows)])

    with jax.named_scope("idx_wait"):
        @pl.when(cid == 0)
        def _():
            pltpu.make_async_copy(pk_hbm.at[pl.ds(sid * C0, C0)],
                                  pk_v, g0).wait()

        @pl.when(cid == 1)
        def _():
            pltpu.make_async_copy(pk_hbm.at[pl.ds(NSUB * C0 + sid * C1, C1)],
                                  pk_v.at[pl.ds(0, C1)], g0).wait()

        plsc.subcore_barrier()

    def _unpack(j, r2, r4):
        for q in range(CHUNK // 16):
            w = pk_v[j, pl.ds(q * 16, 16)]
            sstage[r2, pl.ds(q * 16, 16)] = w & 0xFFFF
            dstage[r4, pl.ds(q * 16, 16)] = lax.shift_right_logical(w, 16)

    def _fire_g(r, b):
        pltpu.async_copy(h_hbm.at[sstage.at[r]], bufs[b], gsem[b])

    def _wait_g(r, b):
        pltpu.make_async_copy(h_hbm.at[sstage.at[r]], bufs[b], gsem[b]).wait()

    def _fire_s(r, b):
        pltpu.async_copy(bufs[b], acc_sh.at[dstage.at[r]], ssem[b], add=True)

    def _wait_s(r, b):
        pltpu.make_async_copy(bufs[b], acc_sh.at[dstage.at[r]], ssem[b]).wait()

    def _emit(j, m2, m4, first, fire_next_g):
        n2 = (m2 + 1) % 2
        if fire_next_g:
            _unpack(j + 1, n2, (m4 + 1) % 4)
            if not first:
                _wait_s((m4 + 3) % 4, n2)
            _fire_g(n2, n2)
        _wait_g(m2, m2)
        _fire_s(m4, m2)

    def _run(nc):
        _unpack(0, 0, 0)
        _fire_g(0, 0)
        _emit(0, 0, 0, True, True)

        steady_n = nc - 2
        loop_n = steady_n // 4
        tail = steady_n % 4

        if loop_n > 0:
            def _steady(g, carry):
                j = 1 + 4 * g
                for k in range(4):
                    _emit(j + k, (1 + k) % 2, (1 + k) % 4, False, True)
                return carry

            lax.fori_loop(0, loop_n, _steady, 0)
        for t in range(tail):
            j = 1 + 4 * loop_n + t
            _emit(j, j % 2, j % 4, False, True)

        j = nc - 1
        _emit(j, j % 2, j % 4, False, False)
        _wait_s((nc - 2) % 4, (nc - 2) % 2)
        _wait_s((nc - 1) % 4, (nc - 1) % 2)

    with jax.named_scope("edge_pipe"):
        @pl.when(cid == 0)
        def _():
            _run(C0)

        @pl.when(cid == 1)
        def _():
            _run(C1)

        plsc.subcore_barrier()

    with jax.named_scope("out_copy"):
        @pl.when(cid == 0)
        def _():
            pltpu.sync_copy(acc_sh.at[pl.ds(base, ROWS_PER_TILE)],
                            out0_hbm.at[pl.ds(base, ROWS_PER_TILE)])

        @pl.when(cid == 1)
        def _():
            pltpu.sync_copy(acc_sh.at[pl.ds(base, ROWS_PER_TILE)],
                            out1_hbm.at[pl.ds(base, ROWS_PER_TILE)])


ER = E // CHUNK
_PBLK = 1280


def _pack_body(s_ref, d_ref, o_ref):
    i = pl.program_id(0)
    grow = i * _PBLK + lax.broadcasted_iota(jnp.int32, (_PBLK, CHUNK), 0)
    col = lax.broadcasted_iota(jnp.int32, (_PBLK, CHUNK), 1)
    pe = (grow - ER) * CHUNK + col
    pad_val = pe | ((N + pe % (ACC_ROWS - N)) << 16)
    real_val = s_ref[0] | (d_ref[0] << 16)
    o_ref[...] = jnp.where(grow < ER, real_val, pad_val)


_pack = pl.pallas_call(
    _pack_body,
    grid=(TOTAL_CHUNKS // _PBLK,),
    in_specs=[
        pl.BlockSpec((1, _PBLK, CHUNK), lambda i: (0, i, 0)),
        pl.BlockSpec((1, _PBLK, CHUNK), lambda i: (1, i, 0)),
    ],
    out_specs=pl.BlockSpec((_PBLK, CHUNK), lambda i: (i, 0)),
    out_shape=jax.ShapeDtypeStruct((TOTAL_CHUNKS, CHUNK), jnp.int32),
)


def _mlp_body(x_ref, a0_ref, a1_ref, w1_ref, b1_ref, w2_ref, b2_ref, o_ref):
    h = x_ref[...] + a0_ref[...] + a1_ref[...]
    y = jnp.dot(h, w1_ref[...], preferred_element_type=jnp.float32) + b1_ref[...]
    y = jnp.maximum(y, 0.0)
    z = jnp.dot(y, w2_ref[...], preferred_element_type=jnp.float32) + b2_ref[...]
    o_ref[...] = jnp.maximum(z, 0.0)


_BLK = 2000
_mlp = pl.pallas_call(
    _mlp_body,
    grid=(N // _BLK,),
    in_specs=[
        pl.BlockSpec((_BLK, NHID), lambda i: (i, 0)),
        pl.BlockSpec((_BLK, NHID), lambda i: (i, 0)),
        pl.BlockSpec((_BLK, NHID), lambda i: (i, 0)),
        pl.BlockSpec((NHID, NHID), lambda i: (0, 0)),
        pl.BlockSpec((1, NHID), lambda i: (0, 0)),
        pl.BlockSpec((NHID, NHID), lambda i: (0, 0)),
        pl.BlockSpec((1, NHID), lambda i: (0, 0)),
    ],
    out_specs=pl.BlockSpec((_BLK, NHID), lambda i: (i, 0)),
    out_shape=jax.ShapeDtypeStruct((N, NHID), jnp.float32),
)


def kernel(x, edge_index, W1_0, b1_0, W2_0, b2_0, W1_1, b1_1, W2_1, b2_1):
    ei3 = edge_index.reshape(2, ER, CHUNK)
    packed = _pack(ei3, ei3)

    h = x
    for (W1, b1, W2, b2) in ((W1_0, b1_0, W2_0, b2_0), (W1_1, b1_1, W2_1, b2_1)):
        a0, a1 = _sc_aggregate(h, packed)
        h = _mlp(h, a0, a1, W1, b1.reshape(1, NHID), W2, b2.reshape(1, NHID))
    return h
